# baseline (speedup 1.0000x reference)
import functools

import jax
import jax.numpy as jnp
from jax import lax
from jax.experimental import pallas as pl
from jax.experimental.pallas import tpu as pltpu
from jax.experimental.pallas import tpu_sc as plsc

N = 10000
E = 320000
D = 128
EPS_MSG = 1e-7
EPS_BN = 1e-5

NUM_CORES = 2
NUM_TILES = 16
E_PER_TILE = E // NUM_TILES
CHUNK = 128
NUM_CHUNKS = E_PER_TILE // CHUNK
TAIL = E_PER_TILE - NUM_CHUNKS * CHUNK
ROWS_PER_TILE = 624
ROWS_REM = N - NUM_TILES * ROWS_PER_TILE


def _tables_body(x_ref, t_ref, tbl_ref):
    x = x_ref[...]
    t = t_ref[0]
    m = jnp.maximum(x, 0.0) + EPS_MSG
    logits = m * t
    c = jnp.max(logits, axis=0, keepdims=True)
    ex = jnp.exp(logits - c)
    tbl_ref[0:N, :] = ex
    tbl_ref[N:2 * N, :] = m * ex


def _make_tables(x, t):
    return pl.pallas_call(
        _tables_body,
        out_shape=jax.ShapeDtypeStruct((2 * N, D), jnp.float32),
        in_specs=[
            pl.BlockSpec(memory_space=pltpu.VMEM),
            pl.BlockSpec(memory_space=pltpu.SMEM),
        ],
        out_specs=pl.BlockSpec(memory_space=pltpu.VMEM),
    )(x, t.reshape((1,)))


_SC_MESH = plsc.VectorSubcoreMesh(core_axis_name="c", subcore_axis_name="s")


@functools.partial(
    pl.kernel,
    out_type=[
        jax.ShapeDtypeStruct((N, D), jnp.float32),
        jax.ShapeDtypeStruct((N, D), jnp.float32),
    ],
    mesh=_SC_MESH,
    scratch_types=(
        [pltpu.VMEM((CHUNK,), jnp.int32) for _ in range(3)] +
        [pltpu.VMEM((CHUNK,), jnp.int32) for _ in range(3)] +
        [
            pltpu.VMEM((TAIL,), jnp.int32),
            pltpu.VMEM((TAIL,), jnp.int32),
        ] +
        [pltpu.VMEM((CHUNK, D), jnp.float32) for _ in range(3)] +
        [pltpu.VMEM_SHARED((N, D), jnp.float32)] +
        [pltpu.SemaphoreType.DMA for _ in range(3)] +
        [pltpu.SemaphoreType.DMA for _ in range(3)]
    ),
)
def _sc_scatter(tbl, src_e, dst_e, zeros_hbm, sum_ex, sum_mex,
                si0, si1, si2, di0, di1, di2, src_it, dst_it,
                r0, r1, r2, acc_sh,
                is0, is1, is2, gs0, gs1, gs2):
    src_ib = [si0, si1, si2]
    dst_ib = [di0, di1, di2]
    rows = [r0, r1, r2]
    isems = [is0, is1, is2]
    gsems = [gs0, gs1, gs2]
    c = lax.axis_index("c")
    s = lax.axis_index("s")
    row0 = s * ROWS_PER_TILE
    for j in range(ROWS_PER_TILE // CHUNK):
        pltpu.sync_copy(zeros_hbm, acc_sh.at[pl.ds(row0 + j * CHUNK, CHUNK)])
    _rem0 = ROWS_PER_TILE % CHUNK
    if _rem0:
        pltpu.sync_copy(
            zeros_hbm.at[pl.ds(0, _rem0)],
            acc_sh.at[pl.ds(row0 + ROWS_PER_TILE - _rem0, _rem0)])

    @pl.when(s == 0)
    def _():
        pltpu.sync_copy(zeros_hbm.at[pl.ds(0, ROWS_REM)],
                        acc_sh.at[pl.ds(NUM_TILES * ROWS_PER_TILE, ROWS_REM)])

    plsc.subcore_barrier()
    ebase = s * E_PER_TILE
    tblc = tbl.at[pl.ds(c * N, N)]

    def issue_idx(k, j, sem):
        pltpu.async_copy(src_e.at[pl.ds(ebase + k * CHUNK, CHUNK)],
                         src_ib[j], sem)
        pltpu.async_copy(dst_e.at[pl.ds(ebase + k * CHUNK, CHUNK)],
                         dst_ib[j], sem)

    def wait_idx(k, j, sem):
        pltpu.make_async_copy(src_e.at[pl.ds(ebase + k * CHUNK, CHUNK)],
                              src_ib[j], sem).wait()
        pltpu.make_async_copy(dst_e.at[pl.ds(ebase + k * CHUNK, CHUNK)],
                              dst_ib[j], sem).wait()

    for j in range(3):
        issue_idx(j, j, isems[j])
    for j in range(2):
        wait_idx(j, j, isems[j])
        pltpu.async_copy(tblc.at[src_ib[j]], rows[j], gsems[j])

    def step(i, j):
        pltpu.make_async_copy(tblc.at[src_ib[j]], rows[j], gsems[j]).wait()

        @pl.when(i + 2 < NUM_CHUNKS)
        def _():
            j2 = (j + 2) % 3
            wait_idx(i + 2, j2, isems[j2])
            pltpu.async_copy(tblc.at[src_ib[j2]], rows[j2], gsems[j2])

        pltpu.sync_copy(rows[j], acc_sh.at[dst_ib[j]], add=True)

        @pl.when(i + 3 < NUM_CHUNKS)
        def _():
            issue_idx(i + 3, j, isems[j])

    def body(k, carry):
        i0 = 3 * k
        for j in range(3):
            step(i0 + j, j)
        return carry

    lax.fori_loop(0, NUM_CHUNKS // 3, body, 0)

    toff = ebase + NUM_CHUNKS * CHUNK
    pltpu.sync_copy(src_e.at[pl.ds(toff, TAIL)], src_it)
    pltpu.sync_copy(dst_e.at[pl.ds(toff, TAIL)], dst_it)
    pltpu.async_copy(tblc.at[src_it], r0.at[pl.ds(0, TAIL)], gs0).wait()
    pltpu.sync_copy(r0.at[pl.ds(0, TAIL)], acc_sh.at[dst_it], add=True)
    plsc.subcore_barrier()

    @pl.when(c == 0)
    def _():
        pltpu.sync_copy(acc_sh.at[pl.ds(row0, ROWS_PER_TILE)],
                        sum_ex.at[pl.ds(row0, ROWS_PER_TILE)])

        @pl.when(s == 0)
        def _():
            pltpu.sync_copy(
                acc_sh.at[pl.ds(NUM_TILES * ROWS_PER_TILE, ROWS_REM)],
                sum_ex.at[pl.ds(NUM_TILES * ROWS_PER_TILE, ROWS_REM)])

    @pl.when(c == 1)
    def _():
        pltpu.sync_copy(acc_sh.at[pl.ds(row0, ROWS_PER_TILE)],
                        sum_mex.at[pl.ds(row0, ROWS_PER_TILE)])

        @pl.when(s == 0)
        def _():
            pltpu.sync_copy(
                acc_sh.at[pl.ds(NUM_TILES * ROWS_PER_TILE, ROWS_REM)],
                sum_mex.at[pl.ds(NUM_TILES * ROWS_PER_TILE, ROWS_REM)])


def _mlp_body(x_ref, se_ref, sm_ref, w1_ref, w2_ref, g_ref, b_ref,
              rm_ref, rv_ref, o_ref):
    x = x_ref[...]
    agg = sm_ref[...] / (se_ref[...] + 1e-16)
    out = agg + x
    scale = g_ref[...] * lax.rsqrt(rv_ref[...] + EPS_BN)
    bias = b_ref[...] - rm_ref[...] * scale
    h = jnp.dot(out, w1_ref[...], preferred_element_type=jnp.float32)
    h = jnp.maximum(h * scale + bias, 0.0)
    o_ref[...] = x + jnp.dot(h, w2_ref[...], preferred_element_type=jnp.float32)


def _mlp(x, sum_ex, sum_mex, w1, w2, gamma, beta, rm, rv):
    nb = 2000
    grid = N // nb
    row_spec = pl.BlockSpec((nb, D), lambda i: (i, 0))
    full = lambda shape: pl.BlockSpec(shape, lambda i: (0,) * len(shape))
    h = w1.shape[1]
    return pl.pallas_call(
        _mlp_body,
        grid=(grid,),
        out_shape=jax.ShapeDtypeStruct((N, D), jnp.float32),
        in_specs=[
            row_spec, row_spec, row_spec,
            full((D, h)), full((h, D)),
            full((1, h)), full((1, h)), full((1, h)), full((1, h)),
        ],
        out_specs=row_spec,
    )(x, sum_ex, sum_mex, w1, w2,
      gamma.reshape(1, h), beta.reshape(1, h), rm.reshape(1, h),
      rv.reshape(1, h))


def kernel(x, edge_index, batch, w1, w2, gamma, beta, running_mean,
           running_var, t):
    tbl = _make_tables(x, t)
    zeros = jnp.zeros((CHUNK, D), jnp.float32)
    sum_ex, sum_mex = _sc_scatter(tbl, edge_index[0], edge_index[1], zeros)
    return _mlp(x, sum_ex, sum_mex, w1, w2, gamma, beta, running_mean,
                running_var)
